# baseline (speedup 1.0000x reference)
import functools

import jax
import jax.numpy as jnp
from jax import lax
from jax.experimental import pallas as pl
from jax.experimental.pallas import tpu as pltpu
from jax.experimental.pallas import tpu_sc as plsc

_B, _C, _N, _K = 4, 128, 2048, 9
_RK = 1024
_TK = _N // _RK
_RN = 256
_T = _N // _RN
_F32 = jnp.float32

_NC, _NS = 2, 16
_NW = _NC * _NS
_GTOT = _B * _N * _K
_CH = 128
_NCH = _GTOT // _CH // _NW
_SLAB = 24


def _dot(a, b):
    return lax.dot_general(a, b, (((1,), (0,)), ((), ())),
                           preferred_element_type=_F32)


def _topk_body(xt_ref, xa_ref, idx_ref):
    b = pl.program_id(0)
    xt = xt_ref[0]
    xa = xa_ref[0]
    s = lax.dot_general(xt, xa, (((1,), (1,)), ((), ())),
                        preferred_element_type=_F32)
    s3 = s.reshape(_RK // 128, 128, _N)
    xt3 = xt.reshape(_RK // 128, 128, _C)
    xx_t = jnp.sum(xt3 * xt3, axis=2, keepdims=True)
    xx_a = jnp.sum(xa * xa, axis=1)
    p = (-xx_t + 2.0 * s3) - xx_a[None, None, :]
    col = lax.broadcasted_iota(jnp.int32, p.shape, 2)
    base = b * _N
    for i in range(_K):
        m = jnp.max(p, axis=2, keepdims=True)
        amax = jnp.min(jnp.where(p == m, col, _N), axis=2)
        idx_ref[i] = amax + base
        p = jnp.where(col == amax[:, :, None], -jnp.inf, p)


def _topk_indices(x_t):
    return pl.pallas_call(
        _topk_body,
        grid=(_B, _TK),
        in_specs=[
            pl.BlockSpec((1, _RK, _C), lambda b, t: (b, t, 0)),
            pl.BlockSpec((1, _N, _C), lambda b, t: (b, 0, 0)),
        ],
        out_specs=pl.BlockSpec((_K, _RK // 128, 128),
                               lambda b, t: (0, b * _TK + t, 0)),
        out_shape=jax.ShapeDtypeStruct((_K, _B * _N // 128, 128), jnp.int32),
        compiler_params=pltpu.CompilerParams(
            dimension_semantics=("parallel", "arbitrary")),
    )(x_t, x_t)


def _gather_body(table_ref, idx_ref, out_ref, idxv, rows, sem0, sem1):
    cid = lax.axis_index("c")
    sid = lax.axis_index("s")
    wid = sid * _NC + cid
    base = wid * _NCH
    aligned = (base // 8) * 8
    loc = base - aligned
    pltpu.sync_copy(idx_ref.at[pl.ds(aligned, _SLAB)], idxv)
    sems = [sem0, sem1]
    cp = pltpu.async_copy(table_ref.at[idxv.at[loc]], rows.at[0], sem0)
    for ci in range(_NCH):
        cur = ci % 2
        cp.wait()
        if ci + 1 < _NCH:
            cp = pltpu.async_copy(table_ref.at[idxv.at[loc + ci + 1]],
                                  rows.at[1 - cur], sems[1 - cur])
        pltpu.sync_copy(rows.at[cur],
                        out_ref.at[pl.ds((base + ci) * _CH, _CH)])


def _gather_rows(table, idx2d):
    mesh = plsc.VectorSubcoreMesh(core_axis_name="c", subcore_axis_name="s")
    run = pl.kernel(
        _gather_body,
        out_type=jax.ShapeDtypeStruct((_GTOT, _C), _F32),
        mesh=mesh,
        scratch_types=[
            pltpu.VMEM((_SLAB, _CH), jnp.int32),
            pltpu.VMEM((2, _CH, _C), _F32),
            pltpu.SemaphoreType.DMA,
            pltpu.SemaphoreType.DMA,
        ],
    )
    return run(table, idx2d)


def _conv1_body(x_ref, f_ref, wx_ref, wd_ref, b1_ref, y_ref, st_ref):
    x = x_ref[0]
    xw = _dot(x, wx_ref[...]) + b1_ref[0][None, :]
    acc_s = jnp.zeros((_C,), _F32)
    acc_q = jnp.zeros((_C,), _F32)
    for p_ in range(3):
        y = xw
        for j in range(3):
            q = 3 * p_ + j
            d = x - f_ref[q, 0]
            dd = d - jnp.abs(d)
            y = y + _dot(dd, wd_ref[j])
        y_ref[0, :, p_ * _C:(p_ + 1) * _C] = y
        acc_s = acc_s + jnp.sum(y, axis=0)
        acc_q = acc_q + jnp.sum(y * y, axis=0)
    st_ref[0, 0, 0, :] = acc_s
    st_ref[0, 0, 1, :] = acc_q


def _conv1(x_t, feat, wx, wd, b1):
    return pl.pallas_call(
        _conv1_body,
        grid=(_B, _T),
        in_specs=[
            pl.BlockSpec((1, _RN, _C), lambda b, t: (b, t, 0)),
            pl.BlockSpec((_K, 1, _RN, _C), lambda b, t: (0, b, t, 0)),
            pl.BlockSpec((_C, _C), lambda b, t: (0, 0)),
            pl.BlockSpec((3, _C, _C), lambda b, t: (0, 0, 0)),
            pl.BlockSpec((1, _C), lambda b, t: (0, 0)),
        ],
        out_specs=[
            pl.BlockSpec((1, _RN, 3 * _C), lambda b, t: (b, t, 0)),
            pl.BlockSpec((1, 1, 2, _C), lambda b, t: (b, t, 0, 0)),
        ],
        out_shape=[
            jax.ShapeDtypeStruct((_B, _N, 3 * _C), _F32),
            jax.ShapeDtypeStruct((_B, _T, 2, _C), _F32),
        ],
        compiler_params=pltpu.CompilerParams(
            dimension_semantics=("parallel", "arbitrary")),
    )(x_t, feat, wx, wd, b1)


def _conv2_body(y1_ref, st_ref, g1_ref, be1_ref, w2_ref, b2_ref,
                y2_ref, st2_ref):
    st = st_ref[...]
    cnt = _F32(_B * _N * 3)
    s = jnp.sum(st[:, :, 0, :], axis=(0, 1))
    q = jnp.sum(st[:, :, 1, :], axis=(0, 1))
    mean = s / cnt
    var = q / cnt - mean * mean
    sc = g1_ref[0] * lax.rsqrt(var + 1e-5)
    sh = be1_ref[0] - mean * sc
    y1 = y1_ref[0]
    y = jnp.broadcast_to(b2_ref[0][None, :], (_RN, _C))
    for j in range(3):
        z = y1[:, j * _C:(j + 1) * _C] * sc[None, :] + sh[None, :]
        z = jnp.maximum(z, 0.0)
        y = y + _dot(z, w2_ref[j])
    y2_ref[0] = y
    st2_ref[0, 0, 0, :] = jnp.sum(y, axis=0)
    st2_ref[0, 0, 1, :] = jnp.sum(y * y, axis=0)


def _conv2(y1, st1, g1, be1, w2, b2):
    return pl.pallas_call(
        _conv2_body,
        grid=(_B, _T),
        in_specs=[
            pl.BlockSpec((1, _RN, 3 * _C), lambda b, t: (b, t, 0)),
            pl.BlockSpec((_B, _T, 2, _C), lambda b, t: (0, 0, 0, 0)),
            pl.BlockSpec((1, _C), lambda b, t: (0, 0)),
            pl.BlockSpec((1, _C), lambda b, t: (0, 0)),
            pl.BlockSpec((3, _C, _C), lambda b, t: (0, 0, 0)),
            pl.BlockSpec((1, _C), lambda b, t: (0, 0)),
        ],
        out_specs=[
            pl.BlockSpec((1, _RN, _C), lambda b, t: (b, t, 0)),
            pl.BlockSpec((1, 1, 2, _C), lambda b, t: (b, t, 0, 0)),
        ],
        out_shape=[
            jax.ShapeDtypeStruct((_B, _N, _C), _F32),
            jax.ShapeDtypeStruct((_B, _T, 2, _C), _F32),
        ],
        compiler_params=pltpu.CompilerParams(
            dimension_semantics=("parallel", "arbitrary")),
    )(y1, st1, g1, be1, w2, b2)


def _bn2_body(y2_ref, st_ref, g2_ref, be2_ref, out_ref):
    st = st_ref[...]
    cnt = _F32(_B * _N)
    s = jnp.sum(st[:, :, 0, :], axis=(0, 1))
    q = jnp.sum(st[:, :, 1, :], axis=(0, 1))
    mean = s / cnt
    var = q / cnt - mean * mean
    sc = g2_ref[0] * lax.rsqrt(var + 1e-5)
    sh = be2_ref[0] - mean * sc
    z = jnp.maximum(y2_ref[0] * sc[None, :] + sh[None, :], 0.0)
    out_ref[0] = z.T


def _bn2(y2, st2, g2, be2):
    return pl.pallas_call(
        _bn2_body,
        grid=(_B, _T),
        in_specs=[
            pl.BlockSpec((1, _RN, _C), lambda b, t: (b, t, 0)),
            pl.BlockSpec((_B, _T, 2, _C), lambda b, t: (0, 0, 0, 0)),
            pl.BlockSpec((1, _C), lambda b, t: (0, 0)),
            pl.BlockSpec((1, _C), lambda b, t: (0, 0)),
        ],
        out_specs=pl.BlockSpec((1, _C, _RN), lambda b, t: (b, 0, t)),
        out_shape=jax.ShapeDtypeStruct((_B, _C, _N), _F32),
        compiler_params=pltpu.CompilerParams(
            dimension_semantics=("parallel", "arbitrary")),
    )(y2, st2, g2, be2)


@jax.jit
def kernel(features, W1, b1, g1, be1, W2, b2, g2, be2):
    x_t = jnp.transpose(features.reshape(_B, _C, _N), (0, 2, 1))

    idx = _topk_indices(x_t)
    idx2d = idx.reshape(_GTOT // _CH, _CH)
    table = x_t.reshape(_B * _N, _C)
    feat = _gather_rows(table, idx2d)
    feat = feat.reshape(_K, _B, _N, _C)

    w1 = W1.reshape(_C, 2 * _C, 3)
    wx = jnp.transpose(jnp.sum(w1[:, :_C, :], axis=2))
    wd = jnp.transpose(w1[:, _C:, :], (2, 1, 0))
    y1, st1 = _conv1(x_t, feat, wx, wd, b1.reshape(1, _C))

    w2 = jnp.transpose(W2.reshape(_C, _C, 3), (2, 1, 0))
    y2, st2 = _conv2(y1, st1,
                     g1.reshape(1, _C), be1.reshape(1, _C),
                     w2, b2.reshape(1, _C))

    out = _bn2(y2, st2, g2.reshape(1, _C), be2.reshape(1, _C))
    return out[:, :, :, None]


# speedup vs baseline: 1.0713x; 1.0713x over previous
import functools

import jax
import jax.numpy as jnp
from jax import lax
from jax.experimental import pallas as pl
from jax.experimental.pallas import tpu as pltpu
from jax.experimental.pallas import tpu_sc as plsc

_B, _C, _N, _K = 4, 128, 2048, 9
_BH = _B // 2
_RK = 1024
_TK = _N // _RK
_RN = 256
_T = _N // _RN
_F32 = jnp.float32

_NC, _NS = 2, 16
_NW = _NC * _NS
_GH = _BH * _N * _K
_CH = 128
_NCH = _GH // _CH // _NW
_SLAB = 16


def _dot(a, b):
    return lax.dot_general(a, b, (((1,), (0,)), ((), ())),
                           preferred_element_type=_F32)


def _topk_body(xt_ref, xa_ref, idx_ref, *, b_off):
    b = pl.program_id(0)
    xt = xt_ref[0]
    xa = xa_ref[0]
    s = lax.dot_general(xt, xa, (((1,), (1,)), ((), ())),
                        preferred_element_type=_F32)
    s3 = s.reshape(_RK // 128, 128, _N)
    xt3 = xt.reshape(_RK // 128, 128, _C)
    xx_t = jnp.sum(xt3 * xt3, axis=2, keepdims=True)
    xx_a = jnp.sum(xa * xa, axis=1)
    p = (-xx_t + 2.0 * s3) - xx_a[None, None, :]
    col = lax.broadcasted_iota(jnp.int32, p.shape, 2)
    base = (b + b_off) * _N
    for i in range(_K):
        m = jnp.max(p, axis=2, keepdims=True)
        amax = jnp.min(jnp.where(p == m, col, _N), axis=2)
        idx_ref[i] = amax + base
        if i + 1 < _K:
            p = jnp.where(col == amax[:, :, None], -jnp.inf, p)


def _topk_half(x_t, b_off):
    return pl.pallas_call(
        functools.partial(_topk_body, b_off=b_off),
        grid=(_BH, _TK),
        in_specs=[
            pl.BlockSpec((1, _RK, _C), lambda b, t: (b + b_off, t, 0)),
            pl.BlockSpec((1, _N, _C), lambda b, t: (b + b_off, 0, 0)),
        ],
        out_specs=pl.BlockSpec((_K, _RK // 128, 128),
                               lambda b, t: (0, b * _TK + t, 0)),
        out_shape=jax.ShapeDtypeStruct((_K, _BH * _N // 128, 128), jnp.int32),
        compiler_params=pltpu.CompilerParams(
            dimension_semantics=("parallel", "arbitrary")),
    )(x_t, x_t)


def _gather_body(table_ref, idx_ref, out_ref, idxv, rows, sem0, sem1):
    cid = lax.axis_index("c")
    sid = lax.axis_index("s")
    wid = sid * _NC + cid
    base = wid * _NCH
    aligned = (base // 8) * 8
    loc = base - aligned
    pltpu.sync_copy(idx_ref.at[pl.ds(aligned, _SLAB)], idxv)
    sems = [sem0, sem1]
    cp = pltpu.async_copy(table_ref.at[idxv.at[loc]], rows.at[0], sem0)
    for ci in range(_NCH):
        cur = ci % 2
        cp.wait()
        if ci + 1 < _NCH:
            cp = pltpu.async_copy(table_ref.at[idxv.at[loc + ci + 1]],
                                  rows.at[1 - cur], sems[1 - cur])
        pltpu.sync_copy(rows.at[cur],
                        out_ref.at[pl.ds((base + ci) * _CH, _CH)])


def _gather_rows(table, idx2d):
    mesh = plsc.VectorSubcoreMesh(core_axis_name="c", subcore_axis_name="s")
    run = pl.kernel(
        _gather_body,
        out_type=jax.ShapeDtypeStruct((_GH, _C), _F32),
        mesh=mesh,
        scratch_types=[
            pltpu.VMEM((_SLAB, _CH), jnp.int32),
            pltpu.VMEM((2, _CH, _C), _F32),
            pltpu.SemaphoreType.DMA,
            pltpu.SemaphoreType.DMA,
        ],
    )
    return run(table, idx2d)


def _conv1_body(x_ref, f_ref, wx_ref, wd_ref, b1_ref, y_ref, st_ref):
    x = x_ref[0]
    xw = _dot(x, wx_ref[...]) + b1_ref[0][None, :]
    acc_s = jnp.zeros((_C,), _F32)
    acc_q = jnp.zeros((_C,), _F32)
    for p_ in range(3):
        y = xw
        for j in range(3):
            q = 3 * p_ + j
            d = x - f_ref[q, 0]
            dd = d - jnp.abs(d)
            y = y + _dot(dd, wd_ref[j])
        y_ref[0, :, p_ * _C:(p_ + 1) * _C] = y
        acc_s = acc_s + jnp.sum(y, axis=0)
        acc_q = acc_q + jnp.sum(y * y, axis=0)
    st_ref[0, 0, 0, :] = acc_s
    st_ref[0, 0, 1, :] = acc_q


def _conv1_half(x_t, feat, wx, wd, b1, b_off):
    return pl.pallas_call(
        _conv1_body,
        grid=(_BH, _T),
        in_specs=[
            pl.BlockSpec((1, _RN, _C), lambda b, t: (b + b_off, t, 0)),
            pl.BlockSpec((_K, 1, _RN, _C), lambda b, t: (0, b, t, 0)),
            pl.BlockSpec((_C, _C), lambda b, t: (0, 0)),
            pl.BlockSpec((3, _C, _C), lambda b, t: (0, 0, 0)),
            pl.BlockSpec((1, _C), lambda b, t: (0, 0)),
        ],
        out_specs=[
            pl.BlockSpec((1, _RN, 3 * _C), lambda b, t: (b, t, 0)),
            pl.BlockSpec((1, 1, 2, _C), lambda b, t: (b, t, 0, 0)),
        ],
        out_shape=[
            jax.ShapeDtypeStruct((_BH, _N, 3 * _C), _F32),
            jax.ShapeDtypeStruct((_BH, _T, 2, _C), _F32),
        ],
        compiler_params=pltpu.CompilerParams(
            dimension_semantics=("parallel", "arbitrary")),
    )(x_t, feat, wx, wd, b1)


def _conv2_body(y1_ref, sta_ref, stb_ref, g1_ref, be1_ref, w2_ref, b2_ref,
                y2_ref, st2_ref):
    sta = sta_ref[...]
    stb = stb_ref[...]
    cnt = _F32(_B * _N * 3)
    s = jnp.sum(sta[:, :, 0, :], axis=(0, 1)) + jnp.sum(stb[:, :, 0, :], axis=(0, 1))
    q = jnp.sum(sta[:, :, 1, :], axis=(0, 1)) + jnp.sum(stb[:, :, 1, :], axis=(0, 1))
    mean = s / cnt
    var = q / cnt - mean * mean
    sc = g1_ref[0] * lax.rsqrt(var + 1e-5)
    sh = be1_ref[0] - mean * sc
    y1 = y1_ref[0]
    y = jnp.broadcast_to(b2_ref[0][None, :], (_RN, _C))
    for j in range(3):
        z = y1[:, j * _C:(j + 1) * _C] * sc[None, :] + sh[None, :]
        z = jnp.maximum(z, 0.0)
        y = y + _dot(z, w2_ref[j])
    y2_ref[0] = y
    st2_ref[0, 0, 0, :] = jnp.sum(y, axis=0)
    st2_ref[0, 0, 1, :] = jnp.sum(y * y, axis=0)


def _conv2_half(y1, st1a, st1b, g1, be1, w2, b2):
    return pl.pallas_call(
        _conv2_body,
        grid=(_BH, _T),
        in_specs=[
            pl.BlockSpec((1, _RN, 3 * _C), lambda b, t: (b, t, 0)),
            pl.BlockSpec((_BH, _T, 2, _C), lambda b, t: (0, 0, 0, 0)),
            pl.BlockSpec((_BH, _T, 2, _C), lambda b, t: (0, 0, 0, 0)),
            pl.BlockSpec((1, _C), lambda b, t: (0, 0)),
            pl.BlockSpec((1, _C), lambda b, t: (0, 0)),
            pl.BlockSpec((3, _C, _C), lambda b, t: (0, 0, 0)),
            pl.BlockSpec((1, _C), lambda b, t: (0, 0)),
        ],
        out_specs=[
            pl.BlockSpec((1, _RN, _C), lambda b, t: (b, t, 0)),
            pl.BlockSpec((1, 1, 2, _C), lambda b, t: (b, t, 0, 0)),
        ],
        out_shape=[
            jax.ShapeDtypeStruct((_BH, _N, _C), _F32),
            jax.ShapeDtypeStruct((_BH, _T, 2, _C), _F32),
        ],
        compiler_params=pltpu.CompilerParams(
            dimension_semantics=("parallel", "arbitrary")),
    )(y1, st1a, st1b, g1, be1, w2, b2)


def _bn2_body(y2_ref, sta_ref, stb_ref, g2_ref, be2_ref, out_ref):
    sta = sta_ref[...]
    stb = stb_ref[...]
    cnt = _F32(_B * _N)
    s = jnp.sum(sta[:, :, 0, :], axis=(0, 1)) + jnp.sum(stb[:, :, 0, :], axis=(0, 1))
    q = jnp.sum(sta[:, :, 1, :], axis=(0, 1)) + jnp.sum(stb[:, :, 1, :], axis=(0, 1))
    mean = s / cnt
    var = q / cnt - mean * mean
    sc = g2_ref[0] * lax.rsqrt(var + 1e-5)
    sh = be2_ref[0] - mean * sc
    z = jnp.maximum(y2_ref[0] * sc[None, :] + sh[None, :], 0.0)
    out_ref[0] = z.T


def _bn2_half(y2, st2a, st2b, g2, be2):
    return pl.pallas_call(
        _bn2_body,
        grid=(_BH, _T),
        in_specs=[
            pl.BlockSpec((1, _RN, _C), lambda b, t: (b, t, 0)),
            pl.BlockSpec((_BH, _T, 2, _C), lambda b, t: (0, 0, 0, 0)),
            pl.BlockSpec((_BH, _T, 2, _C), lambda b, t: (0, 0, 0, 0)),
            pl.BlockSpec((1, _C), lambda b, t: (0, 0)),
            pl.BlockSpec((1, _C), lambda b, t: (0, 0)),
        ],
        out_specs=pl.BlockSpec((1, _C, _RN), lambda b, t: (b, 0, t)),
        out_shape=jax.ShapeDtypeStruct((_BH, _C, _N), _F32),
        compiler_params=pltpu.CompilerParams(
            dimension_semantics=("parallel", "arbitrary")),
    )(y2, st2a, st2b, g2, be2)


@jax.jit
def kernel(features, W1, b1, g1, be1, W2, b2, g2, be2):
    x_t = jnp.transpose(features.reshape(_B, _C, _N), (0, 2, 1))
    table = x_t.reshape(_B * _N, _C)

    w1 = W1.reshape(_C, 2 * _C, 3)
    wx = jnp.transpose(jnp.sum(w1[:, :_C, :], axis=2))
    wd = jnp.transpose(w1[:, _C:, :], (2, 1, 0))
    w2 = jnp.transpose(W2.reshape(_C, _C, 3), (2, 1, 0))
    b1r = b1.reshape(1, _C)

    idxs, feats = [], []
    for h in range(2):
        idx = _topk_half(x_t, h * _BH)
        idxs.append(idx)
    for h in range(2):
        idx2d = idxs[h].reshape(_GH // _CH, _CH)
        feat = _gather_rows(table, idx2d)
        feats.append(feat.reshape(_K, _BH, _N, _C))

    y1s, st1s = [], []
    for h in range(2):
        y1, st1 = _conv1_half(x_t, feats[h], wx, wd, b1r, h * _BH)
        y1s.append(y1)
        st1s.append(st1)

    g1r, be1r = g1.reshape(1, _C), be1.reshape(1, _C)
    b2r = b2.reshape(1, _C)
    y2s, st2s = [], []
    for h in range(2):
        y2, st2 = _conv2_half(y1s[h], st1s[0], st1s[1], g1r, be1r, w2, b2r)
        y2s.append(y2)
        st2s.append(st2)

    g2r, be2r = g2.reshape(1, _C), be2.reshape(1, _C)
    outs = [_bn2_half(y2s[h], st2s[0], st2s[1], g2r, be2r) for h in range(2)]
    return jnp.concatenate(outs, axis=0)[:, :, :, None]


# speedup vs baseline: 1.2131x; 1.1323x over previous
import functools

import jax
import jax.numpy as jnp
from jax import lax
from jax.experimental import pallas as pl
from jax.experimental.pallas import tpu as pltpu
from jax.experimental.pallas import tpu_sc as plsc

_B, _C, _N, _K = 4, 128, 2048, 9
_BH = _B // 2
_RK = 1024
_TK = _N // _RK
_RN = 256
_T = _N // _RN
_F32 = jnp.float32

_NC, _NS = 2, 16
_NW = _NC * _NS
_GH = _BH * _N * _K
_CH = 128
_NCH = _GH // _CH // _NW
_SLAB = 16


def _dot(a, b):
    return lax.dot_general(a, b, (((1,), (0,)), ((), ())),
                           preferred_element_type=_F32)


def _topk_body(xt_ref, xa_ref, idx_ref, *, b_off):
    b = pl.program_id(0)
    xt = xt_ref[0]
    xa = xa_ref[0]
    s = lax.dot_general(xt, xa, (((1,), (1,)), ((), ())),
                        preferred_element_type=_F32)
    s3 = s.reshape(_RK // 128, 128, _N)
    xt3 = xt.reshape(_RK // 128, 128, _C)
    xx_t = jnp.sum(xt3 * xt3, axis=2, keepdims=True)
    xx_a = jnp.sum(xa * xa, axis=1)
    p = (-xx_t + 2.0 * s3) - xx_a[None, None, :]
    ncol = -lax.broadcasted_iota(jnp.int32, p.shape, 2).astype(_F32)
    base = (b + b_off) * _N
    for i in range(_K):
        m = jnp.max(p, axis=2, keepdims=True)
        namax = jnp.max(jnp.where(p == m, ncol, -_F32(_N)), axis=2)
        idx_ref[i] = (-namax).astype(jnp.int32) + base
        if i + 1 < _K:
            p = jnp.where(ncol == namax[:, :, None], -jnp.inf, p)


def _topk_half(x_t, b_off):
    return pl.pallas_call(
        functools.partial(_topk_body, b_off=b_off),
        grid=(_BH, _TK),
        in_specs=[
            pl.BlockSpec((1, _RK, _C), lambda b, t: (b + b_off, t, 0)),
            pl.BlockSpec((1, _N, _C), lambda b, t: (b + b_off, 0, 0)),
        ],
        out_specs=pl.BlockSpec((_K, _RK // 128, 128),
                               lambda b, t: (0, b * _TK + t, 0)),
        out_shape=jax.ShapeDtypeStruct((_K, _BH * _N // 128, 128), jnp.int32),
        compiler_params=pltpu.CompilerParams(
            dimension_semantics=("parallel", "arbitrary")),
    )(x_t, x_t)


def _gather_body(table_ref, idx_ref, out_ref, idxv, rows, sem0, sem1):
    cid = lax.axis_index("c")
    sid = lax.axis_index("s")
    wid = sid * _NC + cid
    base = wid * _NCH
    aligned = (base // 8) * 8
    loc = base - aligned
    pltpu.sync_copy(idx_ref.at[pl.ds(aligned, _SLAB)], idxv)
    sems = [sem0, sem1]
    cp = pltpu.async_copy(table_ref.at[idxv.at[loc]], rows.at[0], sem0)
    for ci in range(_NCH):
        cur = ci % 2
        cp.wait()
        if ci + 1 < _NCH:
            cp = pltpu.async_copy(table_ref.at[idxv.at[loc + ci + 1]],
                                  rows.at[1 - cur], sems[1 - cur])
        pltpu.sync_copy(rows.at[cur],
                        out_ref.at[pl.ds((base + ci) * _CH, _CH)])


def _gather_rows(table, idx2d):
    mesh = plsc.VectorSubcoreMesh(core_axis_name="c", subcore_axis_name="s")
    run = pl.kernel(
        _gather_body,
        out_type=jax.ShapeDtypeStruct((_GH, _C), _F32),
        mesh=mesh,
        scratch_types=[
            pltpu.VMEM((_SLAB, _CH), jnp.int32),
            pltpu.VMEM((2, _CH, _C), _F32),
            pltpu.SemaphoreType.DMA,
            pltpu.SemaphoreType.DMA,
        ],
    )
    return run(table, idx2d)


def _conv1_body(x_ref, f_ref, wx_ref, wd_ref, b1_ref, y_ref, st_ref):
    x = x_ref[0]
    xw = _dot(x, wx_ref[...]) + b1_ref[0][None, :]
    acc_s = jnp.zeros((_C,), _F32)
    acc_q = jnp.zeros((_C,), _F32)
    for p_ in range(3):
        y = xw
        for j in range(3):
            q = 3 * p_ + j
            d = x - f_ref[q, 0]
            dd = d - jnp.abs(d)
            y = y + _dot(dd, wd_ref[j])
        y_ref[0, :, p_ * _C:(p_ + 1) * _C] = y
        acc_s = acc_s + jnp.sum(y, axis=0)
        acc_q = acc_q + jnp.sum(y * y, axis=0)
    st_ref[0, 0, 0, :] = acc_s
    st_ref[0, 0, 1, :] = acc_q


def _conv1_half(x_t, feat, wx, wd, b1, b_off):
    return pl.pallas_call(
        _conv1_body,
        grid=(_BH, _T),
        in_specs=[
            pl.BlockSpec((1, _RN, _C), lambda b, t: (b + b_off, t, 0)),
            pl.BlockSpec((_K, 1, _RN, _C), lambda b, t: (0, b, t, 0)),
            pl.BlockSpec((_C, _C), lambda b, t: (0, 0)),
            pl.BlockSpec((3, _C, _C), lambda b, t: (0, 0, 0)),
            pl.BlockSpec((1, _C), lambda b, t: (0, 0)),
        ],
        out_specs=[
            pl.BlockSpec((1, _RN, 3 * _C), lambda b, t: (b, t, 0)),
            pl.BlockSpec((1, 1, 2, _C), lambda b, t: (b, t, 0, 0)),
        ],
        out_shape=[
            jax.ShapeDtypeStruct((_BH, _N, 3 * _C), _F32),
            jax.ShapeDtypeStruct((_BH, _T, 2, _C), _F32),
        ],
        compiler_params=pltpu.CompilerParams(
            dimension_semantics=("parallel", "arbitrary")),
    )(x_t, feat, wx, wd, b1)


def _conv2_body(y1_ref, sta_ref, stb_ref, g1_ref, be1_ref, w2_ref, b2_ref,
                y2_ref, st2_ref):
    sta = sta_ref[...]
    stb = stb_ref[...]
    cnt = _F32(_B * _N * 3)
    s = jnp.sum(sta[:, :, 0, :], axis=(0, 1)) + jnp.sum(stb[:, :, 0, :], axis=(0, 1))
    q = jnp.sum(sta[:, :, 1, :], axis=(0, 1)) + jnp.sum(stb[:, :, 1, :], axis=(0, 1))
    mean = s / cnt
    var = q / cnt - mean * mean
    sc = g1_ref[0] * lax.rsqrt(var + 1e-5)
    sh = be1_ref[0] - mean * sc
    y1 = y1_ref[0]
    y = jnp.broadcast_to(b2_ref[0][None, :], (_RN, _C))
    for j in range(3):
        z = y1[:, j * _C:(j + 1) * _C] * sc[None, :] + sh[None, :]
        z = jnp.maximum(z, 0.0)
        y = y + _dot(z, w2_ref[j])
    y2_ref[0] = y
    st2_ref[0, 0, 0, :] = jnp.sum(y, axis=0)
    st2_ref[0, 0, 1, :] = jnp.sum(y * y, axis=0)


def _conv2_half(y1, st1a, st1b, g1, be1, w2, b2):
    return pl.pallas_call(
        _conv2_body,
        grid=(_BH, _T),
        in_specs=[
            pl.BlockSpec((1, _RN, 3 * _C), lambda b, t: (b, t, 0)),
            pl.BlockSpec((_BH, _T, 2, _C), lambda b, t: (0, 0, 0, 0)),
            pl.BlockSpec((_BH, _T, 2, _C), lambda b, t: (0, 0, 0, 0)),
            pl.BlockSpec((1, _C), lambda b, t: (0, 0)),
            pl.BlockSpec((1, _C), lambda b, t: (0, 0)),
            pl.BlockSpec((3, _C, _C), lambda b, t: (0, 0, 0)),
            pl.BlockSpec((1, _C), lambda b, t: (0, 0)),
        ],
        out_specs=[
            pl.BlockSpec((1, _RN, _C), lambda b, t: (b, t, 0)),
            pl.BlockSpec((1, 1, 2, _C), lambda b, t: (b, t, 0, 0)),
        ],
        out_shape=[
            jax.ShapeDtypeStruct((_BH, _N, _C), _F32),
            jax.ShapeDtypeStruct((_BH, _T, 2, _C), _F32),
        ],
        compiler_params=pltpu.CompilerParams(
            dimension_semantics=("parallel", "arbitrary")),
    )(y1, st1a, st1b, g1, be1, w2, b2)


def _bn2_body(y2_ref, sta_ref, stb_ref, g2_ref, be2_ref, out_ref):
    sta = sta_ref[...]
    stb = stb_ref[...]
    cnt = _F32(_B * _N)
    s = jnp.sum(sta[:, :, 0, :], axis=(0, 1)) + jnp.sum(stb[:, :, 0, :], axis=(0, 1))
    q = jnp.sum(sta[:, :, 1, :], axis=(0, 1)) + jnp.sum(stb[:, :, 1, :], axis=(0, 1))
    mean = s / cnt
    var = q / cnt - mean * mean
    sc = g2_ref[0] * lax.rsqrt(var + 1e-5)
    sh = be2_ref[0] - mean * sc
    z = jnp.maximum(y2_ref[0] * sc[None, :] + sh[None, :], 0.0)
    out_ref[0] = z.T


def _bn2_half(y2, st2a, st2b, g2, be2):
    return pl.pallas_call(
        _bn2_body,
        grid=(_BH, _T),
        in_specs=[
            pl.BlockSpec((1, _RN, _C), lambda b, t: (b, t, 0)),
            pl.BlockSpec((_BH, _T, 2, _C), lambda b, t: (0, 0, 0, 0)),
            pl.BlockSpec((_BH, _T, 2, _C), lambda b, t: (0, 0, 0, 0)),
            pl.BlockSpec((1, _C), lambda b, t: (0, 0)),
            pl.BlockSpec((1, _C), lambda b, t: (0, 0)),
        ],
        out_specs=pl.BlockSpec((1, _C, _RN), lambda b, t: (b, 0, t)),
        out_shape=jax.ShapeDtypeStruct((_BH, _C, _N), _F32),
        compiler_params=pltpu.CompilerParams(
            dimension_semantics=("parallel", "arbitrary")),
    )(y2, st2a, st2b, g2, be2)


@jax.jit
def kernel(features, W1, b1, g1, be1, W2, b2, g2, be2):
    x_t = jnp.transpose(features.reshape(_B, _C, _N), (0, 2, 1))
    table = x_t.reshape(_B * _N, _C)

    w1 = W1.reshape(_C, 2 * _C, 3)
    wx = jnp.transpose(jnp.sum(w1[:, :_C, :], axis=2))
    wd = jnp.transpose(w1[:, _C:, :], (2, 1, 0))
    w2 = jnp.transpose(W2.reshape(_C, _C, 3), (2, 1, 0))
    b1r = b1.reshape(1, _C)

    idxs, feats = [], []
    for h in range(2):
        idx = _topk_half(x_t, h * _BH)
        idxs.append(idx)
    for h in range(2):
        idx2d = idxs[h].reshape(_GH // _CH, _CH)
        feat = _gather_rows(table, idx2d)
        feats.append(feat.reshape(_K, _BH, _N, _C))

    y1s, st1s = [], []
    for h in range(2):
        y1, st1 = _conv1_half(x_t, feats[h], wx, wd, b1r, h * _BH)
        y1s.append(y1)
        st1s.append(st1)

    g1r, be1r = g1.reshape(1, _C), be1.reshape(1, _C)
    b2r = b2.reshape(1, _C)
    y2s, st2s = [], []
    for h in range(2):
        y2, st2 = _conv2_half(y1s[h], st1s[0], st1s[1], g1r, be1r, w2, b2r)
        y2s.append(y2)
        st2s.append(st2)

    g2r, be2r = g2.reshape(1, _C), be2.reshape(1, _C)
    outs = [_bn2_half(y2s[h], st2s[0], st2s[1], g2r, be2r) for h in range(2)]
    return jnp.concatenate(outs, axis=0)[:, :, :, None]
